# baseline (speedup 1.0000x reference)
import functools

import jax
import jax.numpy as jnp
from jax.experimental import pallas as pl
from jax.experimental.pallas import tpu as pltpu

F32 = jnp.float32
BF16 = jnp.bfloat16
HP = None

S, E, NH, NE = 512, 768, 12, 8
HD = E // NH
HID = 4 * E
HBLK = 1024
CELLS = (("fast", 0.1, 5), ("medium", 1.0, 10), ("slow", 10.0, 15))


def _ln0(v, g, b, eps=1e-5):
    mu = jnp.mean(v, axis=0, keepdims=True)
    var = jnp.mean((v - mu) ** 2, axis=0, keepdims=True)
    return (v - mu) / jnp.sqrt(var + eps) * g + b


def _gelu(v):
    return 0.5 * v * (1.0 + jax.lax.erf(v * (2.0 ** -0.5)))


def _attn_body(xT, relT, in_w, in_b, out_w, out_b, n1g, n1b, x1T_o, xc_o,
               qkv_s, o_s):
    xa = xT[...] + relT[...]
    qkv_s[...] = jnp.dot(in_w[...], xa, precision=HP) + in_b[...]

    def head(h, carry):
        qT = qkv_s[pl.ds(HD * h, HD), :]
        kT = qkv_s[pl.ds(E + HD * h, HD), :]
        vT = qkv_s[pl.ds(2 * E + HD * h, HD), :]
        s = jax.lax.dot_general(kT, qT, (((0,), (0,)), ((), ())),
                                precision=HP) / 8.0
        s = s - jnp.max(s, axis=0, keepdims=True)
        p = jnp.exp(s)
        p = p / jnp.sum(p, axis=0, keepdims=True)
        o_s[pl.ds(HD * h, HD), :] = jax.lax.dot_general(
            vT, p, (((1,), (0,)), ((), ())), precision=HP)
        return carry

    jax.lax.fori_loop(0, NH, head, 0)
    attnT = jnp.dot(out_w[...], o_s[...], precision=HP) + out_b[...]
    x1 = _ln0(xT[...] + attnT, n1g[...], n1b[...])
    x1T_o[...] = x1
    xc_o[...] = jnp.mean(x1, axis=1).reshape(1, E)


def _rmatvec(W, v):
    return jnp.sum(W * v, axis=1).reshape(1, -1)


def _ln_row(v, g, b, eps=1e-5):
    mu = jnp.mean(v, axis=1, keepdims=True)
    var = jnp.mean((v - mu) ** 2, axis=1, keepdims=True)
    return (v - mu) / jnp.sqrt(var + eps) * g + b


def _cell_body(tc, cap, xc, h0, cgw, cgb, w1, b1, l1g, l1b, w2, b2, w3, b3,
               tmw, tmb, lng, lnb, h_o):
    xcv = xc[...]
    comp = jax.nn.sigmoid(jnp.dot(cgw[...], xcv, precision=HP) + cgb[...])
    steps = 5 + (jnp.mean(comp) * cap).astype(jnp.int32)
    denom = (steps - 1).astype(F32)
    dt = 1.0 / denom

    def body(i, h):
        hx = jnp.concatenate([h, xcv], axis=0)
        z = jnp.dot(w1[...], hx, precision=HP) + b1[...]
        z = _gelu(_ln0(z, l1g[...], l1b[...]))
        z = _gelu(jnp.dot(w2[...], z, precision=HP) + b2[...])
        dh = jnp.dot(w3[...], z, precision=HP) + b3[...]
        tprev = (i - 1).astype(F32) / denom
        tm = jnp.tanh(tprev * tmw[...] + tmb[...])
        return h + dt * (dh * tm / tc)

    h = jax.lax.fori_loop(1, steps, body, h0[...])
    h_o[...] = _ln0(h, lng[...], lnb[...])


def _router_body(x1T, xc, hf, hm, hs, mixw, mixb, n2g, n2b, rw1, rb1, rw2,
                 rb2, x2T_o, cw_o):
    xcv = xc[...]
    lg = jnp.sum(mixw[...] * xcv, axis=1, keepdims=True) + mixb[...]
    lg = lg - jnp.max(lg, axis=0, keepdims=True)
    wv = jnp.exp(lg)
    wv = wv / jnp.sum(wv, axis=0, keepdims=True)
    comb = wv[0, 0] * hf[...] + wv[1, 0] * hm[...] + wv[2, 0] * hs[...]
    x2 = _ln0(x1T[...] + comb, n2g[...], n2b[...])
    lfT = jnp.broadcast_to(comb, (E, S))
    riT = jnp.concatenate([x2, lfT], axis=0)
    z = _gelu(jnp.dot(rw1[...], riT, precision=HP) + rb1[...])
    logits = jnp.dot(rw2[...], z, precision=HP) + rb2[...]
    eidx = jax.lax.broadcasted_iota(jnp.int32, (NE, S), 0)
    m1 = jnp.max(logits, axis=0, keepdims=True)
    i1 = jnp.min(jnp.where(logits == m1, eidx, NE), axis=0, keepdims=True)
    neg = jnp.where(eidx == i1, -jnp.inf, logits)
    m2 = jnp.max(neg, axis=0, keepdims=True)
    i2 = jnp.min(jnp.where(neg == m2, eidx, NE), axis=0, keepdims=True)
    t1 = jax.nn.sigmoid(m1 - m2)
    t2 = jax.nn.sigmoid(m2 - m1)
    cw_o[...] = (jnp.where(eidx == i1, t1, 0.0)
                 + jnp.where(eidx == i2, t2, 0.0))
    x2T_o[...] = x2


def _expert_body(nblk, last, xfT, accT, cwe, w1, b1, lg, lb, w2, b2, n3g,
                 n3b, *refs):
    if last:
        out_o, y_o, z1s, ssum, ssq, mus, rss, oacc, xfb = refs
    else:
        out_o, z1s, ssum, ssq, mus, rss, oacc, xfb = refs
    g = pl.program_id(0)
    h = nblk * HBLK

    @pl.when(g == 0)
    def _():
        xfb[...] = xfT[...].astype(BF16)

    @pl.when(g < nblk)
    def _():
        z = jnp.dot(w1[...].astype(BF16), xfb[...],
                    preferred_element_type=F32) + b1[...]
        s1 = jnp.sum(z, axis=0, keepdims=True)
        s2 = jnp.sum(z * z, axis=0, keepdims=True)

        @pl.when(g == 0)
        def _():
            ssum[...] = jnp.zeros_like(ssum)
            ssq[...] = jnp.zeros_like(ssq)

        ssum[...] += s1
        ssq[...] += s2
        z1s[pl.ds(g * HBLK, HBLK), :] = z.astype(BF16)

    @pl.when(g >= nblk)
    def _():
        j = g - nblk

        @pl.when(j == 0)
        def _():
            mu = ssum[...] / h
            var = ssq[...] / h - mu * mu
            mus[...] = mu
            rss[...] = jax.lax.rsqrt(var + 1e-5)

        zb = z1s[pl.ds(j * HBLK, HBLK), :].astype(F32)
        ln = (zb - mus[...]) * rss[...] * lg[...] + lb[...]
        gb = _gelu(ln).astype(BF16)
        contrib = jnp.dot(w2[...].astype(BF16), gb, preferred_element_type=F32)

        @pl.when(j == 0)
        def _():
            oacc[...] = jnp.zeros_like(oacc)

        oacc[...] += contrib

        @pl.when(j == nblk - 1)
        def _():
            res = accT[...] + cwe[...] * (oacc[...] + b2[...])
            out_o[...] = res
            if last:
                y_o[...] = _ln0(res, n3g[...], n3b[...])


def _expert_call(e, xfT, accT, cw, p):
    h = HID * (e + 1)
    nblk = h // HBLK
    last = (e == NE - 1)
    w1 = p['e%d_w1' % e]
    w2 = p['e%d_w2' % e]
    b1 = p['e%d_b1' % e].reshape(h, 1)
    lgv = p['e%d_ln_g' % e].reshape(h, 1)
    lbv = p['e%d_ln_b' % e].reshape(h, 1)
    b2 = p['e%d_b2' % e].reshape(E, 1)
    n3g = p['n3_g'].reshape(E, 1)
    n3b = p['n3_b'].reshape(E, 1)
    cwe = cw[e:e + 1]

    full = lambda *shape: pl.BlockSpec(shape, lambda g: (0,) * len(shape))
    a_idx = lambda g: (jnp.minimum(g, nblk - 1), 0)
    b_idx = lambda g: (jnp.clip(g - nblk, 0, nblk - 1), 0)
    in_specs = [
        full(E, S),
        full(E, S),
        full(1, S),
        pl.BlockSpec((HBLK, E), a_idx),
        pl.BlockSpec((HBLK, 1), a_idx),
        pl.BlockSpec((HBLK, 1), b_idx),
        pl.BlockSpec((HBLK, 1), b_idx),
        pl.BlockSpec((E, HBLK), lambda g: (0, jnp.clip(g - nblk, 0, nblk - 1))),
        full(E, 1),
        full(E, 1),
        full(E, 1),
    ]
    out_shape = [jax.ShapeDtypeStruct((E, S), F32)]
    out_specs = [full(E, S)]
    if last:
        out_shape.append(jax.ShapeDtypeStruct((E, S), F32))
        out_specs.append(full(E, S))
    scratch = [
        pltpu.VMEM((h, S), BF16),
        pltpu.VMEM((1, S), F32),
        pltpu.VMEM((1, S), F32),
        pltpu.VMEM((1, S), F32),
        pltpu.VMEM((1, S), F32),
        pltpu.VMEM((E, S), F32),
        pltpu.VMEM((E, S), BF16),
    ]
    return pl.pallas_call(
        functools.partial(_expert_body, nblk, last),
        grid=(2 * nblk,),
        in_specs=in_specs,
        out_specs=out_specs,
        out_shape=out_shape,
        scratch_shapes=scratch,
        compiler_params=pltpu.CompilerParams(
            dimension_semantics=("arbitrary",)),
    )(xfT, accT, cwe, w1, b1, lgv, lbv, w2, b2, n3g, n3b)


def kernel(x, fast_h, medium_h, slow_h, params):
    p = params
    xT = x[0].T
    relT = p['rel_pos'][0].T
    col = lambda a: a.reshape(-1, 1)

    full = lambda *shape: pl.BlockSpec(shape, lambda: (0,) * len(shape))

    x1T, xc = pl.pallas_call(
        _attn_body,
        out_shape=[jax.ShapeDtypeStruct((E, S), F32),
                   jax.ShapeDtypeStruct((1, E), F32)],
        scratch_shapes=[pltpu.VMEM((3 * E, S), F32),
                        pltpu.VMEM((E, S), F32)],
    )(xT, relT, p['in_w'], col(p['in_b']), p['out_w'], col(p['out_b']),
      col(p['n1_g']), col(p['n1_b']))

    xcc = xc.T
    hT = {}
    for c, tc, cap in CELLS:
        h0 = {"fast": fast_h, "medium": medium_h, "slow": slow_h}[c]
        hT[c] = pl.pallas_call(
            functools.partial(_cell_body, tc, cap),
            out_shape=jax.ShapeDtypeStruct((E, 1), F32),
        )(xcc, h0.T, p[c + '_cg_w'], col(p[c + '_cg_b']),
          p[c + '_w1'], col(p[c + '_b1']), col(p[c + '_ln1_g']),
          col(p[c + '_ln1_b']), p[c + '_w2'], col(p[c + '_b2']),
          p[c + '_w3'], col(p[c + '_b3']), p[c + '_tm_w'],
          col(p[c + '_tm_b']), col(p[c + '_ln_g']), col(p[c + '_ln_b']))

    x2T, cw = pl.pallas_call(
        _router_body,
        out_shape=[jax.ShapeDtypeStruct((E, S), F32),
                   jax.ShapeDtypeStruct((NE, S), F32)],
    )(x1T, xc, hT['fast'], hT['medium'], hT['slow'], p['mix_w'],
      col(p['mix_b']), col(p['n2_g']), col(p['n2_b']), p['r_w1'],
      col(p['r_b1']), p['r_w2'], col(p['r_b2']))

    acc = x2T
    for e in range(NE - 1):
        (acc,) = _expert_call(e, x2T, acc, cw, p)
    _, yT = _expert_call(NE - 1, x2T, acc, cw, p)
    return yT.T[None]


# speedup vs baseline: 1.0013x; 1.0013x over previous
import functools

import jax
import jax.numpy as jnp
from jax.experimental import pallas as pl
from jax.experimental.pallas import tpu as pltpu

F32 = jnp.float32
BF16 = jnp.bfloat16
HP = None

S, E, NH, NE = 512, 768, 12, 8
HD = E // NH
HID = 4 * E
HBLK = 1024
CELLS = (("fast", 0.1, 5), ("medium", 1.0, 10), ("slow", 10.0, 15))


def _ln0(v, g, b, eps=1e-5):
    mu = jnp.mean(v, axis=0, keepdims=True)
    var = jnp.mean((v - mu) ** 2, axis=0, keepdims=True)
    return (v - mu) / jnp.sqrt(var + eps) * g + b


def _gelu(v):
    return 0.5 * v * (1.0 + jax.lax.erf(v * (2.0 ** -0.5)))


def _attn_body(xT, relT, in_w, in_b, out_w, out_b, n1g, n1b, x1T_o, xc_o,
               qkv_s, o_s):
    xa = xT[...] + relT[...]
    qkv_s[...] = jnp.dot(in_w[...], xa, precision=HP) + in_b[...]

    def head(h, carry):
        qT = qkv_s[pl.ds(HD * h, HD), :]
        kT = qkv_s[pl.ds(E + HD * h, HD), :]
        vT = qkv_s[pl.ds(2 * E + HD * h, HD), :]
        s = jax.lax.dot_general(kT, qT, (((0,), (0,)), ((), ())),
                                precision=HP) / 8.0
        s = s - jnp.max(s, axis=0, keepdims=True)
        p = jnp.exp(s)
        p = p / jnp.sum(p, axis=0, keepdims=True)
        o_s[pl.ds(HD * h, HD), :] = jax.lax.dot_general(
            vT, p, (((1,), (0,)), ((), ())), precision=HP)
        return carry

    jax.lax.fori_loop(0, NH, head, 0)
    attnT = jnp.dot(out_w[...], o_s[...], precision=HP) + out_b[...]
    x1 = _ln0(xT[...] + attnT, n1g[...], n1b[...])
    x1T_o[...] = x1
    xc_o[...] = jnp.mean(x1, axis=1).reshape(1, E)


def _rmatvec(W, v):
    return jnp.sum(W * v, axis=1).reshape(1, -1)


def _ln_row(v, g, b, eps=1e-5):
    mu = jnp.mean(v, axis=1, keepdims=True)
    var = jnp.mean((v - mu) ** 2, axis=1, keepdims=True)
    return (v - mu) / jnp.sqrt(var + eps) * g + b


def _cell_body(tc, cap, xc, h0, cgw, cgb, w1, b1, l1g, l1b, w2, b2, w3, b3,
               tmw, tmb, lng, lnb, h_o):
    xcv = xc[...]
    comp = jax.nn.sigmoid(jnp.dot(cgw[...], xcv, precision=HP) + cgb[...])
    steps = 5 + (jnp.mean(comp) * cap).astype(jnp.int32)
    denom = (steps - 1).astype(F32)
    dt = 1.0 / denom

    def body(i, h):
        hx = jnp.concatenate([h, xcv], axis=0)
        z = jnp.dot(w1[...], hx, precision=HP) + b1[...]
        z = _gelu(_ln0(z, l1g[...], l1b[...]))
        z = _gelu(jnp.dot(w2[...], z, precision=HP) + b2[...])
        dh = jnp.dot(w3[...], z, precision=HP) + b3[...]
        tprev = (i - 1).astype(F32) / denom
        tm = jnp.tanh(tprev * tmw[...] + tmb[...])
        return h + dt * (dh * tm / tc)

    h = jax.lax.fori_loop(1, steps, body, h0[...])
    h_o[...] = _ln0(h, lng[...], lnb[...])


def _router_body(x1T, xc, hf, hm, hs, mixw, mixb, n2g, n2b, rw1, rb1, rw2,
                 rb2, x2T_o, cw_o):
    xcv = xc[...]
    lg = jnp.sum(mixw[...] * xcv, axis=1, keepdims=True) + mixb[...]
    lg = lg - jnp.max(lg, axis=0, keepdims=True)
    wv = jnp.exp(lg)
    wv = wv / jnp.sum(wv, axis=0, keepdims=True)
    comb = wv[0, 0] * hf[...] + wv[1, 0] * hm[...] + wv[2, 0] * hs[...]
    x2 = _ln0(x1T[...] + comb, n2g[...], n2b[...])
    lfT = jnp.broadcast_to(comb, (E, S))
    riT = jnp.concatenate([x2, lfT], axis=0)
    z = _gelu(jnp.dot(rw1[...], riT, precision=HP) + rb1[...])
    logits = jnp.dot(rw2[...], z, precision=HP) + rb2[...]
    eidx = jax.lax.broadcasted_iota(jnp.int32, (NE, S), 0)
    m1 = jnp.max(logits, axis=0, keepdims=True)
    i1 = jnp.min(jnp.where(logits == m1, eidx, NE), axis=0, keepdims=True)
    neg = jnp.where(eidx == i1, -jnp.inf, logits)
    m2 = jnp.max(neg, axis=0, keepdims=True)
    i2 = jnp.min(jnp.where(neg == m2, eidx, NE), axis=0, keepdims=True)
    t1 = jax.nn.sigmoid(m1 - m2)
    t2 = jax.nn.sigmoid(m2 - m1)
    cw_o[...] = (jnp.where(eidx == i1, t1, 0.0)
                 + jnp.where(eidx == i2, t2, 0.0))
    x2T_o[...] = x2


def _expert_body(nblk, last, xfT, accT, cwe, w1a, w1b, b1, lg, lb, w2a, w2b,
                 b2, n3g, n3b, *refs):
    if last:
        out_o, y_o, z1s, ssum, ssq, mus, rss, oacc, xfb = refs
    else:
        out_o, z1s, ssum, ssq, mus, rss, oacc, xfb = refs
    g = pl.program_id(0)
    h = nblk * HBLK
    HH = HBLK // 2

    @pl.when(g == 0)
    def _():
        xfb[...] = xfT[...].astype(BF16)

    @pl.when(g < nblk)
    def _():
        b1v = b1[...]
        za = jnp.dot(w1a[...].astype(BF16), xfb[...],
                     preferred_element_type=F32) + b1v[:HH]
        zb = jnp.dot(w1b[...].astype(BF16), xfb[...],
                     preferred_element_type=F32) + b1v[HH:]
        s1 = (jnp.sum(za, axis=0, keepdims=True)
              + jnp.sum(zb, axis=0, keepdims=True))
        s2 = (jnp.sum(za * za, axis=0, keepdims=True)
              + jnp.sum(zb * zb, axis=0, keepdims=True))

        @pl.when(g == 0)
        def _():
            ssum[...] = jnp.zeros_like(ssum)
            ssq[...] = jnp.zeros_like(ssq)

        ssum[...] += s1
        ssq[...] += s2
        z1s[pl.ds(g * HBLK, HH), :] = za.astype(BF16)
        z1s[pl.ds(g * HBLK + HH, HH), :] = zb.astype(BF16)

    @pl.when(g >= nblk)
    def _():
        j = g - nblk

        @pl.when(j == 0)
        def _():
            mu = ssum[...] / h
            var = ssq[...] / h - mu * mu
            mus[...] = mu
            rss[...] = jax.lax.rsqrt(var + 1e-5)

        lgv, lbv = lg[...], lb[...]
        zha = z1s[pl.ds(j * HBLK, HH), :].astype(F32)
        zhb = z1s[pl.ds(j * HBLK + HH, HH), :].astype(F32)
        ga = _gelu((zha - mus[...]) * rss[...] * lgv[:HH] + lbv[:HH]).astype(BF16)
        gb = _gelu((zhb - mus[...]) * rss[...] * lgv[HH:] + lbv[HH:]).astype(BF16)
        contrib = (jnp.dot(w2a[...].astype(BF16), ga, preferred_element_type=F32)
                   + jnp.dot(w2b[...].astype(BF16), gb, preferred_element_type=F32))

        @pl.when(j == 0)
        def _():
            oacc[...] = jnp.zeros_like(oacc)

        oacc[...] += contrib

        @pl.when(j == nblk - 1)
        def _():
            res = accT[...] + cwe[...] * (oacc[...] + b2[...])
            out_o[...] = res
            if last:
                y_o[...] = _ln0(res, n3g[...], n3b[...])


def _expert_call(e, xfT, accT, cw, p):
    h = HID * (e + 1)
    nblk = h // HBLK
    last = (e == NE - 1)
    w1 = p['e%d_w1' % e]
    w2 = p['e%d_w2' % e]
    b1 = p['e%d_b1' % e].reshape(h, 1)
    lgv = p['e%d_ln_g' % e].reshape(h, 1)
    lbv = p['e%d_ln_b' % e].reshape(h, 1)
    b2 = p['e%d_b2' % e].reshape(E, 1)
    n3g = p['n3_g'].reshape(E, 1)
    n3b = p['n3_b'].reshape(E, 1)
    cwe = cw[e:e + 1]

    full = lambda *shape: pl.BlockSpec(shape, lambda g: (0,) * len(shape))
    HH = HBLK // 2
    a_idx = lambda g: (jnp.minimum(g, nblk - 1), 0)
    a_lo = lambda g: (2 * jnp.minimum(g, nblk - 1), 0)
    a_hi = lambda g: (2 * jnp.minimum(g, nblk - 1) + 1, 0)
    b_idx = lambda g: (jnp.clip(g - nblk, 0, nblk - 1), 0)
    b_lo = lambda g: (0, 2 * jnp.clip(g - nblk, 0, nblk - 1))
    b_hi = lambda g: (0, 2 * jnp.clip(g - nblk, 0, nblk - 1) + 1)
    in_specs = [
        full(E, S),
        full(E, S),
        full(1, S),
        pl.BlockSpec((HH, E), a_lo),
        pl.BlockSpec((HH, E), a_hi),
        pl.BlockSpec((HBLK, 1), a_idx),
        pl.BlockSpec((HBLK, 1), b_idx),
        pl.BlockSpec((HBLK, 1), b_idx),
        pl.BlockSpec((E, HH), b_lo),
        pl.BlockSpec((E, HH), b_hi),
        full(E, 1),
        full(E, 1),
        full(E, 1),
    ]
    out_shape = [jax.ShapeDtypeStruct((E, S), F32)]
    out_specs = [full(E, S)]
    if last:
        out_shape.append(jax.ShapeDtypeStruct((E, S), F32))
        out_specs.append(full(E, S))
    scratch = [
        pltpu.VMEM((h, S), BF16),
        pltpu.VMEM((1, S), F32),
        pltpu.VMEM((1, S), F32),
        pltpu.VMEM((1, S), F32),
        pltpu.VMEM((1, S), F32),
        pltpu.VMEM((E, S), F32),
        pltpu.VMEM((E, S), BF16),
    ]
    return pl.pallas_call(
        functools.partial(_expert_body, nblk, last),
        grid=(2 * nblk,),
        in_specs=in_specs,
        out_specs=out_specs,
        out_shape=out_shape,
        scratch_shapes=scratch,
        compiler_params=pltpu.CompilerParams(
            dimension_semantics=("arbitrary",)),
    )(xfT, accT, cwe, w1, w1, b1, lgv, lbv, w2, w2, b2, n3g, n3b)


def kernel(x, fast_h, medium_h, slow_h, params):
    p = params
    xT = x[0].T
    relT = p['rel_pos'][0].T
    col = lambda a: a.reshape(-1, 1)

    full = lambda *shape: pl.BlockSpec(shape, lambda: (0,) * len(shape))

    x1T, xc = pl.pallas_call(
        _attn_body,
        out_shape=[jax.ShapeDtypeStruct((E, S), F32),
                   jax.ShapeDtypeStruct((1, E), F32)],
        scratch_shapes=[pltpu.VMEM((3 * E, S), F32),
                        pltpu.VMEM((E, S), F32)],
    )(xT, relT, p['in_w'], col(p['in_b']), p['out_w'], col(p['out_b']),
      col(p['n1_g']), col(p['n1_b']))

    xcc = xc.T
    hT = {}
    for c, tc, cap in CELLS:
        h0 = {"fast": fast_h, "medium": medium_h, "slow": slow_h}[c]
        hT[c] = pl.pallas_call(
            functools.partial(_cell_body, tc, cap),
            out_shape=jax.ShapeDtypeStruct((E, 1), F32),
        )(xcc, h0.T, p[c + '_cg_w'], col(p[c + '_cg_b']),
          p[c + '_w1'], col(p[c + '_b1']), col(p[c + '_ln1_g']),
          col(p[c + '_ln1_b']), p[c + '_w2'], col(p[c + '_b2']),
          p[c + '_w3'], col(p[c + '_b3']), p[c + '_tm_w'],
          col(p[c + '_tm_b']), col(p[c + '_ln_g']), col(p[c + '_ln_b']))

    x2T, cw = pl.pallas_call(
        _router_body,
        out_shape=[jax.ShapeDtypeStruct((E, S), F32),
                   jax.ShapeDtypeStruct((NE, S), F32)],
    )(x1T, xc, hT['fast'], hT['medium'], hT['slow'], p['mix_w'],
      col(p['mix_b']), col(p['n2_g']), col(p['n2_b']), p['r_w1'],
      col(p['r_b1']), p['r_w2'], col(p['r_b2']))

    acc = x2T
    for e in range(NE - 1):
        (acc,) = _expert_call(e, x2T, acc, cw, p)
    _, yT = _expert_call(NE - 1, x2T, acc, cw, p)
    return yT.T[None]


# speedup vs baseline: 1.2003x; 1.1987x over previous
import functools

import jax
import jax.numpy as jnp
from jax.experimental import pallas as pl
from jax.experimental.pallas import tpu as pltpu

F32 = jnp.float32
BF16 = jnp.bfloat16
HP = None

S, E, NH, NE = 512, 768, 12, 8
HD = E // NH
HID = 4 * E
HBLK = 1536
CELLS = (("fast", 0.1, 5), ("medium", 1.0, 10), ("slow", 10.0, 15))


def _ln0(v, g, b, eps=1e-5):
    mu = jnp.mean(v, axis=0, keepdims=True)
    var = jnp.mean((v - mu) ** 2, axis=0, keepdims=True)
    return (v - mu) / jnp.sqrt(var + eps) * g + b


def _gelu(v):
    return 0.5 * v * (1.0 + jax.lax.erf(v * (2.0 ** -0.5)))


def _attn_body(xT, relT, in_w, in_b, out_w, out_b, n1g, n1b, x1T_o, xc_o,
               qkv_s, o_s):
    xa = xT[...] + relT[...]
    qkv_s[...] = jnp.dot(in_w[...], xa, precision=HP) + in_b[...]

    def head(h, carry):
        qT = qkv_s[pl.ds(HD * h, HD), :]
        kT = qkv_s[pl.ds(E + HD * h, HD), :]
        vT = qkv_s[pl.ds(2 * E + HD * h, HD), :]
        s = jax.lax.dot_general(kT, qT, (((0,), (0,)), ((), ())),
                                precision=HP) / 8.0
        s = s - jnp.max(s, axis=0, keepdims=True)
        p = jnp.exp(s)
        p = p / jnp.sum(p, axis=0, keepdims=True)
        o_s[pl.ds(HD * h, HD), :] = jax.lax.dot_general(
            vT, p, (((1,), (0,)), ((), ())), precision=HP)
        return carry

    jax.lax.fori_loop(0, NH, head, 0)
    attnT = jnp.dot(out_w[...], o_s[...], precision=HP) + out_b[...]
    x1 = _ln0(xT[...] + attnT, n1g[...], n1b[...])
    x1T_o[...] = x1
    xc_o[...] = jnp.mean(x1, axis=1).reshape(1, E)


def _rmatvec(W, v):
    return jnp.sum(W * v, axis=1).reshape(1, -1)


def _ln_row(v, g, b, eps=1e-5):
    mu = jnp.mean(v, axis=1, keepdims=True)
    var = jnp.mean((v - mu) ** 2, axis=1, keepdims=True)
    return (v - mu) / jnp.sqrt(var + eps) * g + b


def _cell_body(tc, cap, xc, h0, cgw, cgb, w1, b1, l1g, l1b, w2, b2, w3, b3,
               tmw, tmb, lng, lnb, h_o):
    xcv = xc[...]
    comp = jax.nn.sigmoid(jnp.dot(cgw[...], xcv, precision=HP) + cgb[...])
    steps = 5 + (jnp.mean(comp) * cap).astype(jnp.int32)
    denom = (steps - 1).astype(F32)
    dt = 1.0 / denom

    def body(i, h):
        hx = jnp.concatenate([h, xcv], axis=0)
        z = jnp.dot(w1[...], hx, precision=HP) + b1[...]
        z = _gelu(_ln0(z, l1g[...], l1b[...]))
        z = _gelu(jnp.dot(w2[...], z, precision=HP) + b2[...])
        dh = jnp.dot(w3[...], z, precision=HP) + b3[...]
        tprev = (i - 1).astype(F32) / denom
        tm = jnp.tanh(tprev * tmw[...] + tmb[...])
        return h + dt * (dh * tm / tc)

    h = jax.lax.fori_loop(1, steps, body, h0[...])
    h_o[...] = _ln0(h, lng[...], lnb[...])


def _router_body(x1T, xc, hf, hm, hs, mixw, mixb, n2g, n2b, rw1, rb1, rw2,
                 rb2, x2T_o, cw_o):
    xcv = xc[...]
    lg = jnp.sum(mixw[...] * xcv, axis=1, keepdims=True) + mixb[...]
    lg = lg - jnp.max(lg, axis=0, keepdims=True)
    wv = jnp.exp(lg)
    wv = wv / jnp.sum(wv, axis=0, keepdims=True)
    comb = wv[0, 0] * hf[...] + wv[1, 0] * hm[...] + wv[2, 0] * hs[...]
    x2 = _ln0(x1T[...] + comb, n2g[...], n2b[...])
    lfT = jnp.broadcast_to(comb, (E, S))
    riT = jnp.concatenate([x2, lfT], axis=0)
    z = _gelu(jnp.dot(rw1[...], riT, precision=HP) + rb1[...])
    logits = jnp.dot(rw2[...], z, precision=HP) + rb2[...]
    eidx = jax.lax.broadcasted_iota(jnp.int32, (NE, S), 0)
    m1 = jnp.max(logits, axis=0, keepdims=True)
    i1 = jnp.min(jnp.where(logits == m1, eidx, NE), axis=0, keepdims=True)
    neg = jnp.where(eidx == i1, -jnp.inf, logits)
    m2 = jnp.max(neg, axis=0, keepdims=True)
    i2 = jnp.min(jnp.where(neg == m2, eidx, NE), axis=0, keepdims=True)
    t1 = jax.nn.sigmoid(m1 - m2)
    t2 = jax.nn.sigmoid(m2 - m1)
    cw_o[...] = (jnp.where(eidx == i1, t1, 0.0)
                 + jnp.where(eidx == i2, t2, 0.0))
    x2T_o[...] = x2


def _expert_body(nblk, last, xfT, accT, cwe, w1, b1, lg, lb, w2, b2, n3g,
                 n3b, *refs):
    if last:
        out_o, y_o, z1s, ssum, ssq, mus, rss, oacc, xfb = refs
    else:
        out_o, z1s, ssum, ssq, mus, rss, oacc, xfb = refs
    g = pl.program_id(0)
    h = nblk * HBLK

    @pl.when(g == 0)
    def _():
        xfb[...] = xfT[...].astype(BF16)

    @pl.when(g < nblk)
    def _():
        z = jnp.dot(w1[...].astype(BF16), xfb[...],
                    preferred_element_type=F32) + jnp.transpose(b1[0])
        s1 = jnp.sum(z, axis=0, keepdims=True)
        s2 = jnp.sum(z * z, axis=0, keepdims=True)

        @pl.when(g == 0)
        def _():
            ssum[...] = jnp.zeros_like(ssum)
            ssq[...] = jnp.zeros_like(ssq)

        ssum[...] += s1
        ssq[...] += s2
        z1s[pl.ds(g * HBLK, HBLK), :] = z.astype(BF16)

    @pl.when(g >= nblk)
    def _():
        j = g - nblk

        @pl.when(j == 0)
        def _():
            mu = ssum[...] / h
            var = ssq[...] / h - mu * mu
            mus[...] = mu
            rss[...] = jax.lax.rsqrt(var + 1e-5)

        zb = z1s[pl.ds(j * HBLK, HBLK), :].astype(F32)
        ln = ((zb - mus[...]) * rss[...] * jnp.transpose(lg[0])
              + jnp.transpose(lb[0]))
        gb = _gelu(ln).astype(BF16)
        contrib = jnp.dot(w2[...].astype(BF16), gb, preferred_element_type=F32)

        @pl.when(j == 0)
        def _():
            oacc[...] = jnp.zeros_like(oacc)

        oacc[...] += contrib

        @pl.when(j == nblk - 1)
        def _():
            res = accT[...] + cwe[...] * (oacc[...] + b2[...])
            out_o[...] = res
            if last:
                y_o[...] = _ln0(res, n3g[...], n3b[...])


def _expert_call(e, xfT, accT, cw, p):
    h = HID * (e + 1)
    nblk = h // HBLK
    last = (e == NE - 1)
    w1 = p['e%d_w1' % e]
    w2 = p['e%d_w2' % e]
    b1 = p['e%d_b1' % e].reshape(nblk, 1, HBLK)
    lgv = p['e%d_ln_g' % e].reshape(nblk, 1, HBLK)
    lbv = p['e%d_ln_b' % e].reshape(nblk, 1, HBLK)
    b2 = p['e%d_b2' % e].reshape(E, 1)
    n3g = p['n3_g'].reshape(E, 1)
    n3b = p['n3_b'].reshape(E, 1)
    cwe = cw[e:e + 1]

    full = lambda *shape: pl.BlockSpec(shape, lambda g: (0,) * len(shape))
    a_idx = lambda g: (jnp.minimum(g, nblk - 1), 0)
    b_idx = lambda g: (jnp.clip(g - nblk, 0, nblk - 1), 0)
    in_specs = [
        full(E, S),
        full(E, S),
        full(1, S),
        pl.BlockSpec((HBLK, E), a_idx),
        pl.BlockSpec((1, 1, HBLK), lambda g: (*a_idx(g), 0)),
        pl.BlockSpec((1, 1, HBLK), lambda g: (*b_idx(g), 0)),
        pl.BlockSpec((1, 1, HBLK), lambda g: (*b_idx(g), 0)),
        pl.BlockSpec((E, HBLK), lambda g: (0, jnp.clip(g - nblk, 0, nblk - 1))),
        full(E, 1),
        full(E, 1),
        full(E, 1),
    ]
    out_shape = [jax.ShapeDtypeStruct((E, S), F32)]
    out_specs = [full(E, S)]
    if last:
        out_shape.append(jax.ShapeDtypeStruct((E, S), F32))
        out_specs.append(full(E, S))
    scratch = [
        pltpu.VMEM((h, S), BF16),
        pltpu.VMEM((1, S), F32),
        pltpu.VMEM((1, S), F32),
        pltpu.VMEM((1, S), F32),
        pltpu.VMEM((1, S), F32),
        pltpu.VMEM((E, S), F32),
        pltpu.VMEM((E, S), BF16),
    ]
    return pl.pallas_call(
        functools.partial(_expert_body, nblk, last),
        grid=(2 * nblk,),
        in_specs=in_specs,
        out_specs=out_specs,
        out_shape=out_shape,
        scratch_shapes=scratch,
        compiler_params=pltpu.CompilerParams(
            dimension_semantics=("arbitrary",)),
    )(xfT, accT, cwe, w1, b1, lgv, lbv, w2, b2, n3g, n3b)


def kernel(x, fast_h, medium_h, slow_h, params):
    p = params
    xT = x[0].T
    relT = p['rel_pos'][0].T
    col = lambda a: a.reshape(-1, 1)

    full = lambda *shape: pl.BlockSpec(shape, lambda: (0,) * len(shape))

    x1T, xc = pl.pallas_call(
        _attn_body,
        out_shape=[jax.ShapeDtypeStruct((E, S), F32),
                   jax.ShapeDtypeStruct((1, E), F32)],
        scratch_shapes=[pltpu.VMEM((3 * E, S), F32),
                        pltpu.VMEM((E, S), F32)],
    )(xT, relT, p['in_w'], col(p['in_b']), p['out_w'], col(p['out_b']),
      col(p['n1_g']), col(p['n1_b']))

    xcc = xc.T
    hT = {}
    for c, tc, cap in CELLS:
        h0 = {"fast": fast_h, "medium": medium_h, "slow": slow_h}[c]
        hT[c] = pl.pallas_call(
            functools.partial(_cell_body, tc, cap),
            out_shape=jax.ShapeDtypeStruct((E, 1), F32),
        )(xcc, h0.T, p[c + '_cg_w'], col(p[c + '_cg_b']),
          p[c + '_w1'], col(p[c + '_b1']), col(p[c + '_ln1_g']),
          col(p[c + '_ln1_b']), p[c + '_w2'], col(p[c + '_b2']),
          p[c + '_w3'], col(p[c + '_b3']), p[c + '_tm_w'],
          col(p[c + '_tm_b']), col(p[c + '_ln_g']), col(p[c + '_ln_b']))

    x2T, cw = pl.pallas_call(
        _router_body,
        out_shape=[jax.ShapeDtypeStruct((E, S), F32),
                   jax.ShapeDtypeStruct((NE, S), F32)],
    )(x1T, xc, hT['fast'], hT['medium'], hT['slow'], p['mix_w'],
      col(p['mix_b']), col(p['n2_g']), col(p['n2_b']), p['r_w1'],
      col(p['r_b1']), p['r_w2'], col(p['r_b2']))

    acc = x2T
    for e in range(NE - 1):
        (acc,) = _expert_call(e, x2T, acc, cw, p)
    _, yT = _expert_call(NE - 1, x2T, acc, cw, p)
    return yT.T[None]
